# baseline (speedup 1.0000x reference)
import jax
import jax.numpy as jnp
from jax import lax
from jax.experimental import pallas as pl
from jax.experimental.pallas import tpu as pltpu
from jax.experimental.pallas import tpu_sc as plsc

_FEAT = 64
_SEQ = 200
_BATCH = 4096
_BCOLS = _BATCH // 32


def _make_kernel():
    mesh = plsc.VectorSubcoreMesh(core_axis_name="c", subcore_axis_name="s")

    def body(tok_hbm, tab_hbm, out_hbm, idx_v, rows_v, pane_v, *sems):
        gsem = sems[0:2]
        wsem = sems[2:4]
        wid = lax.axis_index("s") * 2 + lax.axis_index("c")
        b0 = wid * _BCOLS
        pltpu.sync_copy(tok_hbm.at[:, pl.ds(b0, _BCOLS)], idx_v)

        lane = lax.iota(jnp.int32, 16)

        def gather(t, b):
            return pltpu.make_async_copy(
                tab_hbm.at[idx_v.at[t]], rows_v.at[b], gsem[b])

        def write(t, b):
            return pltpu.make_async_copy(
                pane_v.at[b],
                out_hbm.at[pl.ds(t, 1), :, pl.ds(b0, _BCOLS)],
                wsem[b])

        for b in range(2):
            gather(b, b).start()

        def step(t, carry):
            for b in range(2):
                tt = t * 2 + b
                gather(tt, b).wait()

                def per_f(f, c):
                    for k in range(_BCOLS // 16):
                        src = plsc.load_gather(
                            rows_v.at[b], [k * 16 + lane, lane * 0 + f])
                        pane_v[b, 0, f, pl.ds(k * 16, 16)] = src
                    return c

                lax.fori_loop(0, _FEAT, per_f, 0)

                @pl.when(tt + 2 < _SEQ)
                def _():
                    gather(tt + 2, b).start()

                @pl.when(tt >= 2)
                def _():
                    write(tt - 2, b).wait()

                write(tt, b).start()
            return carry

        lax.fori_loop(0, _SEQ // 2, step, 0)
        for b in range(2):
            write(_SEQ - 2 + b, b).wait()

    return pl.kernel(
        body,
        out_type=jax.ShapeDtypeStruct((_SEQ, _FEAT, _BATCH), jnp.float32),
        mesh=mesh,
        compiler_params=pltpu.CompilerParams(
            use_tc_tiling_on_sc=False, needs_layout_passes=False),
        scratch_types=(
            [
                pltpu.VMEM((_SEQ, _BCOLS), jnp.int32),
                pltpu.VMEM((2, _BCOLS, _FEAT), jnp.float32),
                pltpu.VMEM((2, 1, _FEAT, _BCOLS), jnp.float32),
            ]
            + [pltpu.SemaphoreType.DMA] * 4
        ),
    )


def kernel(token_ids, emb_matrix):
    tok_t = token_ids.T.astype(jnp.int32)
    out3 = _make_kernel()(tok_t, emb_matrix)
    return out3.transpose(2, 0, 1)
